# baseline (speedup 1.0000x reference)
import functools

import jax
import jax.numpy as jnp
from jax import lax
from jax.experimental import pallas as pl
from jax.experimental.pallas import tpu as pltpu
from jax.experimental.pallas import tpu_sc as plsc

NC = 2
NS = 16
BR = 256
NBF = 4
_ACC_WORDS = 880_000


def _plan_cg(c, acc_rows):
  cg = min(c, 256)
  while acc_rows * cg > _ACC_WORDS:
    cg //= 2
  return cg


IB = 8
D = 3


def _make_segsum(cg, acc_rows, K, n_pairs):
  rpt = acc_rows // NS
  PB = 2 * IB
  mesh = plsc.VectorSubcoreMesh(
      core_axis_name="c", subcore_axis_name="s", num_cores=NC,
      num_subcores=NS)

  @functools.partial(
      pl.kernel,
      out_type=jax.ShapeDtypeStruct((NC * acc_rows, cg), jnp.float32),
      mesh=mesh,
      scratch_types=(
          [pltpu.VMEM((IB, K), jnp.int32) for _ in range(4)]
          + [pltpu.VMEM((K, cg), jnp.float32) for _ in range(NBF)]
          + [pltpu.VMEM_SHARED((acc_rows, cg), jnp.float32)]
          + [pltpu.SemaphoreType.DMA for _ in range(NBF)]
          + [pltpu.SemaphoreType.DMA, pltpu.SemaphoreType.DMA]
      ),
      compiler_params=pltpu.CompilerParams(use_tc_tiling_on_sc=False),
  )
  def seg(x_hbm, src_hbm, dst_hbm, zero_hbm, out_hbm, *scr):
    srcA, dstA, srcB, dstB = scr[:4]
    rows_bufs = scr[4:4 + NBF]
    acc = scr[4 + NBF]
    sems = scr[5 + NBF:5 + 2 * NBF]
    semA, semB = scr[5 + 2 * NBF:]
    cid = lax.axis_index("c")
    sid = lax.axis_index("s")
    w = cid * NS + sid
    t0 = sid * rpt
    pltpu.sync_copy(zero_hbm.at[pl.ds(t0, rpt)], acc.at[pl.ds(t0, rpt)])
    plsc.subcore_barrier()

    base_row = w * n_pairs * PB

    def load_src(first_row, sb, sem):
      pltpu.async_copy(src_hbm.at[pl.ds(first_row, IB)], sb, sem)

    def load_dst(first_row, db, sem):
      pltpu.async_copy(dst_hbm.at[pl.ds(first_row, IB)], db, sem)

    def load_idx(first_row, sb, db, sem):
      load_src(first_row, sb, sem)
      load_dst(first_row, db, sem)

    def wait_idx(sb, db, sem):
      pltpu.make_async_copy(src_hbm.at[pl.ds(0, IB)], sb, sem).wait()
      pltpu.make_async_copy(dst_hbm.at[pl.ds(0, IB)], db, sem).wait()

    def gather(idx_row, i):
      return pltpu.async_copy(x_hbm.at[idx_row], rows_bufs[i], sems[i])

    @pl.loop(0, n_pairs)
    def _(q):
      row0 = base_row + q * PB
      pltpu.sync_copy(src_hbm.at[pl.ds(row0, IB)], srcA)
      pltpu.sync_copy(dst_hbm.at[pl.ds(row0, IB)], dstA)
      pltpu.sync_copy(src_hbm.at[pl.ds(row0 + IB, IB)], srcB)
      pltpu.sync_copy(dst_hbm.at[pl.ds(row0 + IB, IB)], dstB)
      descs = {}
      for t in range(D):
        descs[t] = gather(srcA.at[t], t % NBF)
      for t in range(PB):
        descs[t].wait()
        tt = t + D
        if tt < PB:
          descs[tt] = gather(
              srcA.at[tt] if tt < IB else srcB.at[tt - IB], tt % NBF)
        drow = dstA.at[t] if t < IB else dstB.at[t - IB]
        pltpu.sync_copy(rows_bufs[t % NBF], acc.at[drow], add=True)

    plsc.subcore_barrier()
    pltpu.sync_copy(acc.at[pl.ds(t0, rpt)],
                    out_hbm.at[pl.ds(cid * acc_rows + t0, rpt)])

  return seg


def _segsum(h_groups, src, dst, n_out, acc_rows):
  cg = h_groups[0].shape[1]
  e = src.shape[0]
  K = min(128, max(8, 8192 // cg))
  n_raw = -(-e // (NC * NS * K))
  n_chunks = _round_up(n_raw, 2 * IB)
  total = n_chunks * NC * NS * K
  pad = total - e
  sp, dp = src, dst
  if pad:
    sp = jnp.concatenate([src, jnp.zeros((pad,), jnp.int32)])
    dp = jnp.concatenate([dst, jnp.full((pad,), n_out, jnp.int32)])
  sp = sp.reshape(n_chunks * NC * NS, K)
  dp = dp.reshape(n_chunks * NC * NS, K)
  extra = jnp.zeros((2 * IB, K), jnp.int32)
  sp = jnp.concatenate([sp, extra])
  dp = jnp.concatenate([dp, extra])
  zeros = jnp.zeros((acc_rows, cg), jnp.float32)
  f = _make_segsum(cg, acc_rows, K, n_chunks // (2 * IB))
  return [f(hg, sp, dp, zeros) for hg in h_groups]


def _mm_stats(parts, W, n_valid, acc_rows):
  cin, cout = W.shape
  G = acc_rows // BR
  ng = len(parts)

  def body(*refs):
    part_refs = refs[:2 * ng]
    w_ref, h_ref, s_ref, q_ref = refs[2 * ng:]
    i = pl.program_id(0)
    s = jnp.concatenate(
        [part_refs[2 * k][...] + part_refs[2 * k + 1][...]
         for k in range(ng)], axis=1)
    h = jnp.dot(s, w_ref[...], preferred_element_type=jnp.float32)
    h_ref[...] = h
    rows = lax.broadcasted_iota(jnp.int32, (BR, 1), 0) + i * BR
    hm = jnp.where(rows < n_valid, h, 0.0)
    ps = jnp.sum(hm, axis=0, keepdims=True)
    pq = jnp.sum(hm * hm, axis=0, keepdims=True)

    @pl.when(i == 0)
    def _():
      s_ref[...] = ps
      q_ref[...] = pq

    @pl.when(i > 0)
    def _():
      s_ref[...] += ps
      q_ref[...] += pq

  in_specs = []
  args = []
  for pk in parts:
    cg = pk.shape[1]
    in_specs.append(pl.BlockSpec((BR, cg), lambda i: (i, 0)))
    in_specs.append(
        pl.BlockSpec((BR, cg), lambda i, _G=G: (i + _G, 0)))
    args.extend([pk, pk])
  in_specs.append(pl.BlockSpec((cin, cout), lambda i: (0, 0)))
  args.append(W)

  return pl.pallas_call(
      body,
      grid=(G,),
      in_specs=in_specs,
      out_specs=[
          pl.BlockSpec((BR, cout), lambda i: (i, 0)),
          pl.BlockSpec((1, cout), lambda i: (0, 0)),
          pl.BlockSpec((1, cout), lambda i: (0, 0)),
      ],
      out_shape=[
          jax.ShapeDtypeStruct((acc_rows, cout), jnp.float32),
          jax.ShapeDtypeStruct((1, cout), jnp.float32),
          jax.ShapeDtypeStruct((1, cout), jnp.float32),
      ],
  )(*args)


def _bn_relu(H, ssum, ssq, g, b, n_valid, cg):
  acc_rows, cout = H.shape
  G = acc_rows // BR
  inv_n = 1.0 / n_valid
  ng = cout // cg

  def body(h_ref, s_ref, q_ref, g_ref, b_ref, *o_refs):
    mu = s_ref[...] * inv_n
    var = q_ref[...] * inv_n - mu * mu
    scale = g_ref[...] * lax.rsqrt(var + 1e-5)
    out = jnp.maximum((h_ref[...] - mu) * scale + b_ref[...], 0.0)
    for k in range(ng):
      o_refs[k][...] = out[:, k * cg:(k + 1) * cg]

  return pl.pallas_call(
      body,
      grid=(G,),
      in_specs=[
          pl.BlockSpec((BR, cout), lambda i: (i, 0)),
          pl.BlockSpec((1, cout), lambda i: (0, 0)),
          pl.BlockSpec((1, cout), lambda i: (0, 0)),
          pl.BlockSpec((1, cout), lambda i: (0, 0)),
          pl.BlockSpec((1, cout), lambda i: (0, 0)),
      ],
      out_specs=[pl.BlockSpec((BR, cg), lambda i: (i, 0))
                 for _ in range(ng)],
      out_shape=[jax.ShapeDtypeStruct((acc_rows, cg), jnp.float32)
                 for _ in range(ng)],
  )(H, ssum, ssq, g, b)


def _bn_gem(H, ssum, ssq, g, b, p, n_valid):
  acc_rows, cout = H.shape
  G = acc_rows // BR
  inv_n = 1.0 / n_valid

  def body(h_ref, s_ref, q_ref, g_ref, b_ref, p_ref, o_ref):
    i = pl.program_id(0)
    mu = s_ref[...] * inv_n
    var = q_ref[...] * inv_n - mu * mu
    scale = g_ref[...] * lax.rsqrt(var + 1e-5)
    h = jnp.maximum((h_ref[...] - mu) * scale + b_ref[...], 0.0)
    f = jnp.maximum(h, 1e-6)
    fp = jnp.exp(p_ref[...] * jnp.log(f))
    rows = lax.broadcasted_iota(jnp.int32, (BR, 1), 0) + i * BR
    fp = jnp.where(rows < n_valid, fp, 0.0)
    ps = jnp.sum(fp, axis=0, keepdims=True)

    @pl.when(i == 0)
    def _():
      o_ref[...] = ps

    @pl.when(i > 0)
    def _():
      o_ref[...] += ps

  return pl.pallas_call(
      body,
      grid=(G,),
      in_specs=[
          pl.BlockSpec((BR, cout), lambda i: (i, 0)),
          pl.BlockSpec((1, cout), lambda i: (0, 0)),
          pl.BlockSpec((1, cout), lambda i: (0, 0)),
          pl.BlockSpec((1, cout), lambda i: (0, 0)),
          pl.BlockSpec((1, cout), lambda i: (0, 0)),
          pl.BlockSpec((1, 1), lambda i: (0, 0)),
      ],
      out_specs=pl.BlockSpec((1, cout), lambda i: (0, 0)),
      out_shape=jax.ShapeDtypeStruct((1, cout), jnp.float32),
  )(H, ssum, ssq, g, b, p)


def _gem_final(gsum, p, n_valid):
  cout = gsum.shape[1]
  inv_n = 1.0 / n_valid

  def body(s_ref, p_ref, o_ref):
    o_ref[...] = jnp.exp(jnp.log(s_ref[...] * inv_n) / p_ref[...])

  return pl.pallas_call(
      body,
      out_shape=jax.ShapeDtypeStruct((1, cout), jnp.float32),
  )(gsum, p)


def _round_up(n, m):
  return -(-n // m) * m


def kernel(x, src0, dst0, src1a, dst1a, src1b, dst1b, src2a, dst2a, src2b,
           dst2b, src3a, dst3a, src3b, dst3b, src4a, dst4a, src4b, dst4b,
           W0, g0, b0, W1a, g1a, b1a, W1b, g1b, b1b, W2a, g2a, b2a, W2b,
           g2b, b2b, W3a, g3a, b3a, W3b, g3b, b3b, W4a, g4a, b4a, W4b,
           g4b, b4b, p):
  h_groups = [jnp.pad(x, ((0, 0), (0, 13)))]
  W0p = jnp.pad(W0, ((0, 13), (0, 0)))

  layers = [
      (src0, dst0, W0p, g0, b0, 25000),
      (src1a, dst1a, W1a, g1a, b1a, 12500),
      (src1b, dst1b, W1b, g1b, b1b, 12500),
      (src2a, dst2a, W2a, g2a, b2a, 6250),
      (src2b, dst2b, W2b, g2b, b2b, 6250),
      (src3a, dst3a, W3a, g3a, b3a, 3125),
      (src3b, dst3b, W3b, g3b, b3b, 3125),
      (src4a, dst4a, W4a, g4a, b4a, 1562),
      (src4b, dst4b, W4b, g4b, b4b, 1562),
  ]
  p2 = p.reshape(1, 1)
  gsum = None
  for li, (src, dst, W, g, b, n_out) in enumerate(layers):
    acc_rows = _round_up(n_out + 1, BR)
    parts = _segsum(h_groups, src, dst, n_out, acc_rows)
    H, ssum, ssq = _mm_stats(parts, W, n_out, acc_rows)
    g2, b2 = g.reshape(1, -1), b.reshape(1, -1)
    if li + 1 == len(layers):
      gsum = _bn_gem(H, ssum, ssq, g2, b2, p2, n_out)
    else:
      next_n_out = layers[li + 1][5]
      next_acc_rows = _round_up(next_n_out + 1, BR)
      cg_next = _plan_cg(W.shape[1], next_acc_rows)
      h_groups = _bn_relu(H, ssum, ssq, g2, b2, n_out, cg_next)
  return _gem_final(gsum, p2, 1562)


# speedup vs baseline: 1.0012x; 1.0012x over previous
import functools

import jax
import jax.numpy as jnp
from jax import lax
from jax.experimental import pallas as pl
from jax.experimental.pallas import tpu as pltpu
from jax.experimental.pallas import tpu_sc as plsc

NC = 2
NS = 16
BR = 256
NBF = 4
_ACC_WORDS = 880_000


def _plan_cg(c, acc_rows):
  cg = min(c, 256)
  while acc_rows * cg > _ACC_WORDS:
    cg //= 2
  return cg


IB = 8
D = 3


def _make_segsum(cg, acc_rows, K, n_pairs):
  rpt = acc_rows // NS
  PB = 2 * IB
  mesh = plsc.VectorSubcoreMesh(
      core_axis_name="c", subcore_axis_name="s", num_cores=NC,
      num_subcores=NS)

  @functools.partial(
      pl.kernel,
      out_type=jax.ShapeDtypeStruct((NC * acc_rows, cg), jnp.float32),
      mesh=mesh,
      scratch_types=(
          [pltpu.VMEM((PB, K), jnp.int32),
           pltpu.VMEM((PB, K), jnp.int32)]
          + [pltpu.VMEM((K, cg), jnp.float32) for _ in range(NBF)]
          + [pltpu.VMEM_SHARED((acc_rows, cg), jnp.float32)]
          + [pltpu.SemaphoreType.DMA for _ in range(NBF)]
      ),
      compiler_params=pltpu.CompilerParams(use_tc_tiling_on_sc=False),
  )
  def seg(x_hbm, src_hbm, dst_hbm, zero_hbm, out_hbm, srcb, dstb, *scr):
    rows_bufs = scr[:NBF]
    acc = scr[NBF]
    sems = scr[NBF + 1:]
    cid = lax.axis_index("c")
    sid = lax.axis_index("s")
    w = cid * NS + sid
    t0 = sid * rpt
    pltpu.sync_copy(zero_hbm.at[pl.ds(t0, rpt)], acc.at[pl.ds(t0, rpt)])
    plsc.subcore_barrier()

    base_row = w * n_pairs * PB

    @pl.loop(0, n_pairs)
    def _(q):
      row0 = base_row + q * PB
      pltpu.sync_copy(src_hbm.at[pl.ds(row0, PB)], srcb)
      pltpu.sync_copy(dst_hbm.at[pl.ds(row0, PB)], dstb)
      descs = {}
      for t in range(D):
        descs[t] = pltpu.async_copy(
            x_hbm.at[srcb.at[t]], rows_bufs[t % NBF], sems[t % NBF])
      for t in range(PB):
        descs[t].wait()
        tt = t + D
        if tt < PB:
          descs[tt] = pltpu.async_copy(
              x_hbm.at[srcb.at[tt]], rows_bufs[tt % NBF], sems[tt % NBF])
        pltpu.sync_copy(rows_bufs[t % NBF], acc.at[dstb.at[t]], add=True)

    plsc.subcore_barrier()
    pltpu.sync_copy(acc.at[pl.ds(t0, rpt)],
                    out_hbm.at[pl.ds(cid * acc_rows + t0, rpt)])

  return seg


def _segsum(h_groups, src, dst, n_out, acc_rows):
  cg = h_groups[0].shape[1]
  e = src.shape[0]
  K = min(128, max(8, 8192 // cg))
  n_raw = -(-e // (NC * NS * K))
  n_chunks = _round_up(n_raw, 2 * IB)
  total = n_chunks * NC * NS * K
  pad = total - e
  sp, dp = src, dst
  if pad:
    sp = jnp.concatenate([src, jnp.zeros((pad,), jnp.int32)])
    dp = jnp.concatenate([dst, jnp.full((pad,), n_out, jnp.int32)])
  sp = sp.reshape(n_chunks * NC * NS, K)
  dp = dp.reshape(n_chunks * NC * NS, K)
  extra = jnp.zeros((2 * IB, K), jnp.int32)
  sp = jnp.concatenate([sp, extra])
  dp = jnp.concatenate([dp, extra])
  zeros = jnp.zeros((acc_rows, cg), jnp.float32)
  f = _make_segsum(cg, acc_rows, K, n_chunks // (2 * IB))
  return [f(hg, sp, dp, zeros) for hg in h_groups]


def _mm_stats(parts, W, n_valid, acc_rows):
  cin, cout = W.shape
  G = acc_rows // BR
  ng = len(parts)

  def body(*refs):
    part_refs = refs[:2 * ng]
    w_ref, h_ref, s_ref, q_ref = refs[2 * ng:]
    i = pl.program_id(0)
    s = jnp.concatenate(
        [part_refs[2 * k][...] + part_refs[2 * k + 1][...]
         for k in range(ng)], axis=1)
    h = jnp.dot(s, w_ref[...], preferred_element_type=jnp.float32)
    h_ref[...] = h
    rows = lax.broadcasted_iota(jnp.int32, (BR, 1), 0) + i * BR
    hm = jnp.where(rows < n_valid, h, 0.0)
    ps = jnp.sum(hm, axis=0, keepdims=True)
    pq = jnp.sum(hm * hm, axis=0, keepdims=True)

    @pl.when(i == 0)
    def _():
      s_ref[...] = ps
      q_ref[...] = pq

    @pl.when(i > 0)
    def _():
      s_ref[...] += ps
      q_ref[...] += pq

  in_specs = []
  args = []
  for pk in parts:
    cg = pk.shape[1]
    in_specs.append(pl.BlockSpec((BR, cg), lambda i: (i, 0)))
    in_specs.append(
        pl.BlockSpec((BR, cg), lambda i, _G=G: (i + _G, 0)))
    args.extend([pk, pk])
  in_specs.append(pl.BlockSpec((cin, cout), lambda i: (0, 0)))
  args.append(W)

  return pl.pallas_call(
      body,
      grid=(G,),
      in_specs=in_specs,
      out_specs=[
          pl.BlockSpec((BR, cout), lambda i: (i, 0)),
          pl.BlockSpec((1, cout), lambda i: (0, 0)),
          pl.BlockSpec((1, cout), lambda i: (0, 0)),
      ],
      out_shape=[
          jax.ShapeDtypeStruct((acc_rows, cout), jnp.float32),
          jax.ShapeDtypeStruct((1, cout), jnp.float32),
          jax.ShapeDtypeStruct((1, cout), jnp.float32),
      ],
  )(*args)


def _bn_relu(H, ssum, ssq, g, b, n_valid, cg):
  acc_rows, cout = H.shape
  G = acc_rows // BR
  inv_n = 1.0 / n_valid
  ng = cout // cg

  def body(h_ref, s_ref, q_ref, g_ref, b_ref, *o_refs):
    mu = s_ref[...] * inv_n
    var = q_ref[...] * inv_n - mu * mu
    scale = g_ref[...] * lax.rsqrt(var + 1e-5)
    out = jnp.maximum((h_ref[...] - mu) * scale + b_ref[...], 0.0)
    for k in range(ng):
      o_refs[k][...] = out[:, k * cg:(k + 1) * cg]

  return pl.pallas_call(
      body,
      grid=(G,),
      in_specs=[
          pl.BlockSpec((BR, cout), lambda i: (i, 0)),
          pl.BlockSpec((1, cout), lambda i: (0, 0)),
          pl.BlockSpec((1, cout), lambda i: (0, 0)),
          pl.BlockSpec((1, cout), lambda i: (0, 0)),
          pl.BlockSpec((1, cout), lambda i: (0, 0)),
      ],
      out_specs=[pl.BlockSpec((BR, cg), lambda i: (i, 0))
                 for _ in range(ng)],
      out_shape=[jax.ShapeDtypeStruct((acc_rows, cg), jnp.float32)
                 for _ in range(ng)],
  )(H, ssum, ssq, g, b)


def _bn_gem(H, ssum, ssq, g, b, p, n_valid):
  acc_rows, cout = H.shape
  G = acc_rows // BR
  inv_n = 1.0 / n_valid

  def body(h_ref, s_ref, q_ref, g_ref, b_ref, p_ref, o_ref):
    i = pl.program_id(0)
    mu = s_ref[...] * inv_n
    var = q_ref[...] * inv_n - mu * mu
    scale = g_ref[...] * lax.rsqrt(var + 1e-5)
    h = jnp.maximum((h_ref[...] - mu) * scale + b_ref[...], 0.0)
    f = jnp.maximum(h, 1e-6)
    fp = jnp.exp(p_ref[...] * jnp.log(f))
    rows = lax.broadcasted_iota(jnp.int32, (BR, 1), 0) + i * BR
    fp = jnp.where(rows < n_valid, fp, 0.0)
    ps = jnp.sum(fp, axis=0, keepdims=True)

    @pl.when(i == 0)
    def _():
      o_ref[...] = ps

    @pl.when(i > 0)
    def _():
      o_ref[...] += ps

  return pl.pallas_call(
      body,
      grid=(G,),
      in_specs=[
          pl.BlockSpec((BR, cout), lambda i: (i, 0)),
          pl.BlockSpec((1, cout), lambda i: (0, 0)),
          pl.BlockSpec((1, cout), lambda i: (0, 0)),
          pl.BlockSpec((1, cout), lambda i: (0, 0)),
          pl.BlockSpec((1, cout), lambda i: (0, 0)),
          pl.BlockSpec((1, 1), lambda i: (0, 0)),
      ],
      out_specs=pl.BlockSpec((1, cout), lambda i: (0, 0)),
      out_shape=jax.ShapeDtypeStruct((1, cout), jnp.float32),
  )(H, ssum, ssq, g, b, p)


def _gem_final(gsum, p, n_valid):
  cout = gsum.shape[1]
  inv_n = 1.0 / n_valid

  def body(s_ref, p_ref, o_ref):
    o_ref[...] = jnp.exp(jnp.log(s_ref[...] * inv_n) / p_ref[...])

  return pl.pallas_call(
      body,
      out_shape=jax.ShapeDtypeStruct((1, cout), jnp.float32),
  )(gsum, p)


def _round_up(n, m):
  return -(-n // m) * m


def kernel(x, src0, dst0, src1a, dst1a, src1b, dst1b, src2a, dst2a, src2b,
           dst2b, src3a, dst3a, src3b, dst3b, src4a, dst4a, src4b, dst4b,
           W0, g0, b0, W1a, g1a, b1a, W1b, g1b, b1b, W2a, g2a, b2a, W2b,
           g2b, b2b, W3a, g3a, b3a, W3b, g3b, b3b, W4a, g4a, b4a, W4b,
           g4b, b4b, p):
  h_groups = [jnp.pad(x, ((0, 0), (0, 13)))]
  W0p = jnp.pad(W0, ((0, 13), (0, 0)))

  layers = [
      (src0, dst0, W0p, g0, b0, 25000),
      (src1a, dst1a, W1a, g1a, b1a, 12500),
      (src1b, dst1b, W1b, g1b, b1b, 12500),
      (src2a, dst2a, W2a, g2a, b2a, 6250),
      (src2b, dst2b, W2b, g2b, b2b, 6250),
      (src3a, dst3a, W3a, g3a, b3a, 3125),
      (src3b, dst3b, W3b, g3b, b3b, 3125),
      (src4a, dst4a, W4a, g4a, b4a, 1562),
      (src4b, dst4b, W4b, g4b, b4b, 1562),
  ]
  p2 = p.reshape(1, 1)
  gsum = None
  for li, (src, dst, W, g, b, n_out) in enumerate(layers):
    acc_rows = _round_up(n_out + 1, BR)
    parts = _segsum(h_groups, src, dst, n_out, acc_rows)
    H, ssum, ssq = _mm_stats(parts, W, n_out, acc_rows)
    g2, b2 = g.reshape(1, -1), b.reshape(1, -1)
    if li + 1 == len(layers):
      gsum = _bn_gem(H, ssum, ssq, g2, b2, p2, n_out)
    else:
      next_n_out = layers[li + 1][5]
      next_acc_rows = _round_up(next_n_out + 1, BR)
      cg_next = _plan_cg(W.shape[1], next_acc_rows)
      h_groups = _bn_relu(H, ssum, ssq, g2, b2, n_out, cg_next)
  return _gem_final(gsum, p2, 1562)


# speedup vs baseline: 3.0342x; 3.0305x over previous
import functools

import jax
import jax.numpy as jnp
from jax import lax
from jax.experimental import pallas as pl
from jax.experimental.pallas import tpu as pltpu
from jax.experimental.pallas import tpu_sc as plsc

NC = 2
NS = 16
BR = 256
NBF = 4
_ACC_WORDS = 880_000


def _plan_cg(c, acc_rows):
  cg = min(c, 256)
  while acc_rows * cg > _ACC_WORDS:
    cg //= 2
  return cg


def _make_segsum(cg, acc_rows, K, PB, n_pairs):
  rpt = acc_rows // NS
  D = min(3, PB - 1)
  mesh = plsc.VectorSubcoreMesh(
      core_axis_name="c", subcore_axis_name="s", num_cores=NC,
      num_subcores=NS)

  @functools.partial(
      pl.kernel,
      out_type=jax.ShapeDtypeStruct((NC * acc_rows, cg), jnp.float32),
      mesh=mesh,
      scratch_types=(
          [pltpu.VMEM((PB, K), jnp.int32),
           pltpu.VMEM((PB, K), jnp.int32)]
          + [pltpu.VMEM((K, cg), jnp.float32) for _ in range(NBF)]
          + [pltpu.VMEM_SHARED((acc_rows, cg), jnp.float32)]
          + [pltpu.SemaphoreType.DMA for _ in range(NBF)]
      ),
      compiler_params=pltpu.CompilerParams(use_tc_tiling_on_sc=False),
  )
  def seg(x_hbm, src_hbm, dst_hbm, zero_hbm, out_hbm, srcb, dstb, *scr):
    rows_bufs = scr[:NBF]
    acc = scr[NBF]
    sems = scr[NBF + 1:]
    cid = lax.axis_index("c")
    sid = lax.axis_index("s")
    w = cid * NS + sid
    t0 = sid * rpt
    pltpu.sync_copy(zero_hbm.at[pl.ds(t0, rpt)], acc.at[pl.ds(t0, rpt)])
    plsc.subcore_barrier()

    base_row = w * n_pairs * PB

    @pl.loop(0, n_pairs)
    def _(q):
      row0 = base_row + q * PB
      pltpu.sync_copy(src_hbm.at[pl.ds(row0, PB)], srcb)
      pltpu.sync_copy(dst_hbm.at[pl.ds(row0, PB)], dstb)
      descs = {}
      for t in range(D):
        descs[t] = pltpu.async_copy(
            x_hbm.at[srcb.at[t]], rows_bufs[t % NBF], sems[t % NBF])
      for t in range(PB):
        descs[t].wait()
        tt = t + D
        if tt < PB:
          descs[tt] = pltpu.async_copy(
              x_hbm.at[srcb.at[tt]], rows_bufs[tt % NBF], sems[tt % NBF])
        pltpu.sync_copy(rows_bufs[t % NBF], acc.at[dstb.at[t]], add=True)

    plsc.subcore_barrier()
    pltpu.sync_copy(acc.at[pl.ds(t0, rpt)],
                    out_hbm.at[pl.ds(cid * acc_rows + t0, rpt)])

  return seg


def _segsum(h_groups, src, dst, n_out, acc_rows):
  cg = h_groups[0].shape[1]
  e = src.shape[0]
  K = min(128, max(8, 8192 // cg))
  n_raw = -(-e // (NC * NS * K))
  n_chunks, neg_ib = min((_round_up(n_raw, ib), -ib) for ib in (16, 8, 4))
  pb = -neg_ib
  total = n_chunks * NC * NS * K
  pad = total - e
  sp, dp = src, dst
  if pad:
    sp = jnp.concatenate([src, jnp.zeros((pad,), jnp.int32)])
    dp = jnp.concatenate([dst, jnp.full((pad,), n_out, jnp.int32)])
  sp = sp.reshape(n_chunks * NC * NS, K)
  dp = dp.reshape(n_chunks * NC * NS, K)
  zeros = jnp.zeros((acc_rows, cg), jnp.float32)
  f = _make_segsum(cg, acc_rows, K, pb, n_chunks // pb)
  return [f(hg, sp, dp, zeros) for hg in h_groups]


def _mm_stats(parts, W, n_valid, acc_rows):
  cin, cout = W.shape
  G = acc_rows // BR
  ng = len(parts)

  def body(*refs):
    part_refs = refs[:2 * ng]
    w_ref, h_ref, s_ref, q_ref = refs[2 * ng:]
    i = pl.program_id(0)
    s = jnp.concatenate(
        [part_refs[2 * k][...] + part_refs[2 * k + 1][...]
         for k in range(ng)], axis=1)
    h = jnp.dot(s, w_ref[...], preferred_element_type=jnp.float32)
    h_ref[...] = h
    rows = lax.broadcasted_iota(jnp.int32, (BR, 1), 0) + i * BR
    hm = jnp.where(rows < n_valid, h, 0.0)
    ps = jnp.sum(hm, axis=0, keepdims=True)
    pq = jnp.sum(hm * hm, axis=0, keepdims=True)

    @pl.when(i == 0)
    def _():
      s_ref[...] = ps
      q_ref[...] = pq

    @pl.when(i > 0)
    def _():
      s_ref[...] += ps
      q_ref[...] += pq

  in_specs = []
  args = []
  for pk in parts:
    cg = pk.shape[1]
    in_specs.append(pl.BlockSpec((BR, cg), lambda i: (i, 0)))
    in_specs.append(
        pl.BlockSpec((BR, cg), lambda i, _G=G: (i + _G, 0)))
    args.extend([pk, pk])
  in_specs.append(pl.BlockSpec((cin, cout), lambda i: (0, 0)))
  args.append(W)

  return pl.pallas_call(
      body,
      grid=(G,),
      in_specs=in_specs,
      out_specs=[
          pl.BlockSpec((BR, cout), lambda i: (i, 0)),
          pl.BlockSpec((1, cout), lambda i: (0, 0)),
          pl.BlockSpec((1, cout), lambda i: (0, 0)),
      ],
      out_shape=[
          jax.ShapeDtypeStruct((acc_rows, cout), jnp.float32),
          jax.ShapeDtypeStruct((1, cout), jnp.float32),
          jax.ShapeDtypeStruct((1, cout), jnp.float32),
      ],
  )(*args)


def _bn_relu(H, ssum, ssq, g, b, n_valid, cg):
  acc_rows, cout = H.shape
  G = acc_rows // BR
  inv_n = 1.0 / n_valid
  ng = cout // cg

  def body(h_ref, s_ref, q_ref, g_ref, b_ref, *o_refs):
    mu = s_ref[...] * inv_n
    var = q_ref[...] * inv_n - mu * mu
    scale = g_ref[...] * lax.rsqrt(var + 1e-5)
    out = jnp.maximum((h_ref[...] - mu) * scale + b_ref[...], 0.0)
    for k in range(ng):
      o_refs[k][...] = out[:, k * cg:(k + 1) * cg]

  return pl.pallas_call(
      body,
      grid=(G,),
      in_specs=[
          pl.BlockSpec((BR, cout), lambda i: (i, 0)),
          pl.BlockSpec((1, cout), lambda i: (0, 0)),
          pl.BlockSpec((1, cout), lambda i: (0, 0)),
          pl.BlockSpec((1, cout), lambda i: (0, 0)),
          pl.BlockSpec((1, cout), lambda i: (0, 0)),
      ],
      out_specs=[pl.BlockSpec((BR, cg), lambda i: (i, 0))
                 for _ in range(ng)],
      out_shape=[jax.ShapeDtypeStruct((acc_rows, cg), jnp.float32)
                 for _ in range(ng)],
  )(H, ssum, ssq, g, b)


def _bn_gem(H, ssum, ssq, g, b, p, n_valid):
  acc_rows, cout = H.shape
  G = acc_rows // BR
  inv_n = 1.0 / n_valid

  def body(h_ref, s_ref, q_ref, g_ref, b_ref, p_ref, o_ref):
    i = pl.program_id(0)
    mu = s_ref[...] * inv_n
    var = q_ref[...] * inv_n - mu * mu
    scale = g_ref[...] * lax.rsqrt(var + 1e-5)
    h = jnp.maximum((h_ref[...] - mu) * scale + b_ref[...], 0.0)
    f = jnp.maximum(h, 1e-6)
    fp = jnp.exp(p_ref[...] * jnp.log(f))
    rows = lax.broadcasted_iota(jnp.int32, (BR, 1), 0) + i * BR
    fp = jnp.where(rows < n_valid, fp, 0.0)
    ps = jnp.sum(fp, axis=0, keepdims=True)

    @pl.when(i == 0)
    def _():
      o_ref[...] = ps

    @pl.when(i > 0)
    def _():
      o_ref[...] += ps

  return pl.pallas_call(
      body,
      grid=(G,),
      in_specs=[
          pl.BlockSpec((BR, cout), lambda i: (i, 0)),
          pl.BlockSpec((1, cout), lambda i: (0, 0)),
          pl.BlockSpec((1, cout), lambda i: (0, 0)),
          pl.BlockSpec((1, cout), lambda i: (0, 0)),
          pl.BlockSpec((1, cout), lambda i: (0, 0)),
          pl.BlockSpec((1, 1), lambda i: (0, 0)),
      ],
      out_specs=pl.BlockSpec((1, cout), lambda i: (0, 0)),
      out_shape=jax.ShapeDtypeStruct((1, cout), jnp.float32),
  )(H, ssum, ssq, g, b, p)


def _gem_final(gsum, p, n_valid):
  cout = gsum.shape[1]
  inv_n = 1.0 / n_valid

  def body(s_ref, p_ref, o_ref):
    o_ref[...] = jnp.exp(jnp.log(s_ref[...] * inv_n) / p_ref[...])

  return pl.pallas_call(
      body,
      out_shape=jax.ShapeDtypeStruct((1, cout), jnp.float32),
  )(gsum, p)


def _round_up(n, m):
  return -(-n // m) * m


def kernel(x, src0, dst0, src1a, dst1a, src1b, dst1b, src2a, dst2a, src2b,
           dst2b, src3a, dst3a, src3b, dst3b, src4a, dst4a, src4b, dst4b,
           W0, g0, b0, W1a, g1a, b1a, W1b, g1b, b1b, W2a, g2a, b2a, W2b,
           g2b, b2b, W3a, g3a, b3a, W3b, g3b, b3b, W4a, g4a, b4a, W4b,
           g4b, b4b, p):
  h_groups = [jnp.pad(x, ((0, 0), (0, 13)))]
  W0p = jnp.pad(W0, ((0, 13), (0, 0)))

  layers = [
      (src0, dst0, W0p, g0, b0, 25000),
      (src1a, dst1a, W1a, g1a, b1a, 12500),
      (src1b, dst1b, W1b, g1b, b1b, 12500),
      (src2a, dst2a, W2a, g2a, b2a, 6250),
      (src2b, dst2b, W2b, g2b, b2b, 6250),
      (src3a, dst3a, W3a, g3a, b3a, 3125),
      (src3b, dst3b, W3b, g3b, b3b, 3125),
      (src4a, dst4a, W4a, g4a, b4a, 1562),
      (src4b, dst4b, W4b, g4b, b4b, 1562),
  ]
  p2 = p.reshape(1, 1)
  gsum = None
  for li, (src, dst, W, g, b, n_out) in enumerate(layers):
    acc_rows = _round_up(n_out + 1, BR)
    parts = _segsum(h_groups, src, dst, n_out, acc_rows)
    H, ssum, ssq = _mm_stats(parts, W, n_out, acc_rows)
    g2, b2 = g.reshape(1, -1), b.reshape(1, -1)
    if li + 1 == len(layers):
      gsum = _bn_gem(H, ssum, ssq, g2, b2, p2, n_out)
    else:
      next_n_out = layers[li + 1][5]
      next_acc_rows = _round_up(next_n_out + 1, BR)
      cg_next = _plan_cg(W.shape[1], next_acc_rows)
      h_groups = _bn_relu(H, ssum, ssq, g2, b2, n_out, cg_next)
  return _gem_final(gsum, p2, 1562)


# speedup vs baseline: 3.8716x; 1.2760x over previous
import functools

import jax
import jax.numpy as jnp
from jax import lax
from jax.experimental import pallas as pl
from jax.experimental.pallas import tpu as pltpu
from jax.experimental.pallas import tpu_sc as plsc

NC = 2
NS = 16
BR = 256
NBF = 4
_ACC_WORDS = 880_000


def _plan_cg(c, acc_rows):
  for cg in (min(c, 512), 256, 128, 64, 32, 16):
    if cg > c:
      continue
    acc_w = acc_rows * cg
    for K in (128, 64, 32, 16, 8):
      if NBF * K * cg * 4 > 300 * 1024:
        continue
      staging = NS * (NBF * K * cg + 4 * 16 * K + 64)
      if acc_w + staging <= 2_040_000:
        return cg, K
  return 16, 8


def _make_segsum(cg, acc_rows, K, PB, n_pairs):
  rpt = acc_rows // NS
  D = min(3, PB - 1)
  mesh = plsc.VectorSubcoreMesh(
      core_axis_name="c", subcore_axis_name="s", num_cores=NC,
      num_subcores=NS)

  @functools.partial(
      pl.kernel,
      out_type=jax.ShapeDtypeStruct((NC * acc_rows, cg), jnp.float32),
      mesh=mesh,
      scratch_types=(
          [pltpu.VMEM((PB, K), jnp.int32),
           pltpu.VMEM((PB, K), jnp.int32)]
          + [pltpu.VMEM((K, cg), jnp.float32) for _ in range(NBF)]
          + [pltpu.VMEM_SHARED((acc_rows, cg), jnp.float32)]
          + [pltpu.SemaphoreType.DMA for _ in range(NBF)]
      ),
      compiler_params=pltpu.CompilerParams(use_tc_tiling_on_sc=False),
  )
  def seg(x_hbm, src_hbm, dst_hbm, zero_hbm, out_hbm, srcb, dstb, *scr):
    rows_bufs = scr[:NBF]
    acc = scr[NBF]
    sems = scr[NBF + 1:]
    cid = lax.axis_index("c")
    sid = lax.axis_index("s")
    w = cid * NS + sid
    t0 = sid * rpt
    pltpu.sync_copy(zero_hbm.at[pl.ds(t0, rpt)], acc.at[pl.ds(t0, rpt)])
    plsc.subcore_barrier()

    base_row = w * n_pairs * PB

    @pl.loop(0, n_pairs)
    def _(q):
      row0 = base_row + q * PB
      pltpu.sync_copy(src_hbm.at[pl.ds(row0, PB)], srcb)
      pltpu.sync_copy(dst_hbm.at[pl.ds(row0, PB)], dstb)
      descs = {}
      for t in range(D):
        descs[t] = pltpu.async_copy(
            x_hbm.at[srcb.at[t]], rows_bufs[t % NBF], sems[t % NBF])
      for t in range(PB):
        descs[t].wait()
        tt = t + D
        if tt < PB:
          descs[tt] = pltpu.async_copy(
              x_hbm.at[srcb.at[tt]], rows_bufs[tt % NBF], sems[tt % NBF])
        pltpu.sync_copy(rows_bufs[t % NBF], acc.at[dstb.at[t]], add=True)

    plsc.subcore_barrier()
    pltpu.sync_copy(acc.at[pl.ds(t0, rpt)],
                    out_hbm.at[pl.ds(cid * acc_rows + t0, rpt)])

  return seg


def _segsum(h_groups, src, dst, n_out, acc_rows):
  cg = h_groups[0].shape[1]
  e = src.shape[0]
  _, K = _plan_cg(cg, acc_rows)
  n_raw = -(-e // (NC * NS * K))
  n_chunks, neg_ib = min((_round_up(n_raw, ib), -ib) for ib in (16, 8, 4))
  pb = -neg_ib
  total = n_chunks * NC * NS * K
  pad = total - e
  sp, dp = src, dst
  if pad:
    sp = jnp.concatenate([src, jnp.zeros((pad,), jnp.int32)])
    dp = jnp.concatenate([dst, jnp.full((pad,), n_out, jnp.int32)])
  sp = sp.reshape(n_chunks * NC * NS, K)
  dp = dp.reshape(n_chunks * NC * NS, K)
  zeros = jnp.zeros((acc_rows, cg), jnp.float32)
  f = _make_segsum(cg, acc_rows, K, pb, n_chunks // pb)
  return [f(hg, sp, dp, zeros) for hg in h_groups]


def _mm_stats(parts, W, n_valid, acc_rows):
  cin, cout = W.shape
  G = acc_rows // BR
  ng = len(parts)

  def body(*refs):
    part_refs = refs[:2 * ng]
    w_ref, h_ref, s_ref, q_ref = refs[2 * ng:]
    i = pl.program_id(0)
    s = jnp.concatenate(
        [part_refs[2 * k][...] + part_refs[2 * k + 1][...]
         for k in range(ng)], axis=1)
    h = jnp.dot(s, w_ref[...], preferred_element_type=jnp.float32)
    h_ref[...] = h
    rows = lax.broadcasted_iota(jnp.int32, (BR, 1), 0) + i * BR
    hm = jnp.where(rows < n_valid, h, 0.0)
    ps = jnp.sum(hm, axis=0, keepdims=True)
    pq = jnp.sum(hm * hm, axis=0, keepdims=True)

    @pl.when(i == 0)
    def _():
      s_ref[...] = ps
      q_ref[...] = pq

    @pl.when(i > 0)
    def _():
      s_ref[...] += ps
      q_ref[...] += pq

  in_specs = []
  args = []
  for pk in parts:
    cg = pk.shape[1]
    in_specs.append(pl.BlockSpec((BR, cg), lambda i: (i, 0)))
    in_specs.append(
        pl.BlockSpec((BR, cg), lambda i, _G=G: (i + _G, 0)))
    args.extend([pk, pk])
  in_specs.append(pl.BlockSpec((cin, cout), lambda i: (0, 0)))
  args.append(W)

  return pl.pallas_call(
      body,
      grid=(G,),
      in_specs=in_specs,
      out_specs=[
          pl.BlockSpec((BR, cout), lambda i: (i, 0)),
          pl.BlockSpec((1, cout), lambda i: (0, 0)),
          pl.BlockSpec((1, cout), lambda i: (0, 0)),
      ],
      out_shape=[
          jax.ShapeDtypeStruct((acc_rows, cout), jnp.float32),
          jax.ShapeDtypeStruct((1, cout), jnp.float32),
          jax.ShapeDtypeStruct((1, cout), jnp.float32),
      ],
  )(*args)


def _bn_relu(H, ssum, ssq, g, b, n_valid, cg):
  acc_rows, cout = H.shape
  G = acc_rows // BR
  inv_n = 1.0 / n_valid
  ng = cout // cg

  def body(h_ref, s_ref, q_ref, g_ref, b_ref, *o_refs):
    mu = s_ref[...] * inv_n
    var = q_ref[...] * inv_n - mu * mu
    scale = g_ref[...] * lax.rsqrt(var + 1e-5)
    out = jnp.maximum((h_ref[...] - mu) * scale + b_ref[...], 0.0)
    for k in range(ng):
      o_refs[k][...] = out[:, k * cg:(k + 1) * cg]

  return pl.pallas_call(
      body,
      grid=(G,),
      in_specs=[
          pl.BlockSpec((BR, cout), lambda i: (i, 0)),
          pl.BlockSpec((1, cout), lambda i: (0, 0)),
          pl.BlockSpec((1, cout), lambda i: (0, 0)),
          pl.BlockSpec((1, cout), lambda i: (0, 0)),
          pl.BlockSpec((1, cout), lambda i: (0, 0)),
      ],
      out_specs=[pl.BlockSpec((BR, cg), lambda i: (i, 0))
                 for _ in range(ng)],
      out_shape=[jax.ShapeDtypeStruct((acc_rows, cg), jnp.float32)
                 for _ in range(ng)],
  )(H, ssum, ssq, g, b)


def _bn_gem(H, ssum, ssq, g, b, p, n_valid):
  acc_rows, cout = H.shape
  G = acc_rows // BR
  inv_n = 1.0 / n_valid

  def body(h_ref, s_ref, q_ref, g_ref, b_ref, p_ref, o_ref):
    i = pl.program_id(0)
    mu = s_ref[...] * inv_n
    var = q_ref[...] * inv_n - mu * mu
    scale = g_ref[...] * lax.rsqrt(var + 1e-5)
    h = jnp.maximum((h_ref[...] - mu) * scale + b_ref[...], 0.0)
    f = jnp.maximum(h, 1e-6)
    fp = jnp.exp(p_ref[...] * jnp.log(f))
    rows = lax.broadcasted_iota(jnp.int32, (BR, 1), 0) + i * BR
    fp = jnp.where(rows < n_valid, fp, 0.0)
    ps = jnp.sum(fp, axis=0, keepdims=True)

    @pl.when(i == 0)
    def _():
      o_ref[...] = ps

    @pl.when(i > 0)
    def _():
      o_ref[...] += ps

  return pl.pallas_call(
      body,
      grid=(G,),
      in_specs=[
          pl.BlockSpec((BR, cout), lambda i: (i, 0)),
          pl.BlockSpec((1, cout), lambda i: (0, 0)),
          pl.BlockSpec((1, cout), lambda i: (0, 0)),
          pl.BlockSpec((1, cout), lambda i: (0, 0)),
          pl.BlockSpec((1, cout), lambda i: (0, 0)),
          pl.BlockSpec((1, 1), lambda i: (0, 0)),
      ],
      out_specs=pl.BlockSpec((1, cout), lambda i: (0, 0)),
      out_shape=jax.ShapeDtypeStruct((1, cout), jnp.float32),
  )(H, ssum, ssq, g, b, p)


def _gem_final(gsum, p, n_valid):
  cout = gsum.shape[1]
  inv_n = 1.0 / n_valid

  def body(s_ref, p_ref, o_ref):
    o_ref[...] = jnp.exp(jnp.log(s_ref[...] * inv_n) / p_ref[...])

  return pl.pallas_call(
      body,
      out_shape=jax.ShapeDtypeStruct((1, cout), jnp.float32),
  )(gsum, p)


def _round_up(n, m):
  return -(-n // m) * m


def kernel(x, src0, dst0, src1a, dst1a, src1b, dst1b, src2a, dst2a, src2b,
           dst2b, src3a, dst3a, src3b, dst3b, src4a, dst4a, src4b, dst4b,
           W0, g0, b0, W1a, g1a, b1a, W1b, g1b, b1b, W2a, g2a, b2a, W2b,
           g2b, b2b, W3a, g3a, b3a, W3b, g3b, b3b, W4a, g4a, b4a, W4b,
           g4b, b4b, p):
  h_groups = [jnp.pad(x, ((0, 0), (0, 13)))]
  W0p = jnp.pad(W0, ((0, 13), (0, 0)))

  layers = [
      (src0, dst0, W0p, g0, b0, 25000),
      (src1a, dst1a, W1a, g1a, b1a, 12500),
      (src1b, dst1b, W1b, g1b, b1b, 12500),
      (src2a, dst2a, W2a, g2a, b2a, 6250),
      (src2b, dst2b, W2b, g2b, b2b, 6250),
      (src3a, dst3a, W3a, g3a, b3a, 3125),
      (src3b, dst3b, W3b, g3b, b3b, 3125),
      (src4a, dst4a, W4a, g4a, b4a, 1562),
      (src4b, dst4b, W4b, g4b, b4b, 1562),
  ]
  p2 = p.reshape(1, 1)
  gsum = None
  for li, (src, dst, W, g, b, n_out) in enumerate(layers):
    acc_rows = _round_up(n_out + 1, BR)
    parts = _segsum(h_groups, src, dst, n_out, acc_rows)
    H, ssum, ssq = _mm_stats(parts, W, n_out, acc_rows)
    g2, b2 = g.reshape(1, -1), b.reshape(1, -1)
    if li + 1 == len(layers):
      gsum = _bn_gem(H, ssum, ssq, g2, b2, p2, n_out)
    else:
      next_n_out = layers[li + 1][5]
      next_acc_rows = _round_up(next_n_out + 1, BR)
      cg_next = _plan_cg(W.shape[1], next_acc_rows)[0]
      h_groups = _bn_relu(H, ssum, ssq, g2, b2, n_out, cg_next)
  return _gem_final(gsum, p2, 1562)


# speedup vs baseline: 3.9183x; 1.0121x over previous
import functools

import jax
import jax.numpy as jnp
from jax import lax
from jax.experimental import pallas as pl
from jax.experimental.pallas import tpu as pltpu
from jax.experimental.pallas import tpu_sc as plsc

NC = 2
NS = 16
BR = 256
NBF = 4
_ACC_WORDS = 880_000


def _plan_cg(c, acc_rows):
  for cg in (min(c, 512), 256, 128, 64, 32, 16):
    if cg > c:
      continue
    acc_w = acc_rows * cg
    for K in (128, 64, 32, 16, 8):
      if NBF * K * cg * 4 > 300 * 1024:
        continue
      staging = NS * (NBF * K * cg + 4 * 16 * K + 64)
      if acc_w + staging <= 2_040_000:
        return cg, K
  return 16, 8


def _make_segsum(cg, acc_rows, K, PB, n_pairs, n_in, stage):
  rpt = acc_rows // NS
  D = min(3, PB - 1)
  mesh = plsc.VectorSubcoreMesh(
      core_axis_name="c", subcore_axis_name="s", num_cores=NC,
      num_subcores=NS)

  @functools.partial(
      pl.kernel,
      out_type=jax.ShapeDtypeStruct((NC * acc_rows, cg), jnp.float32),
      mesh=mesh,
      scratch_types=(
          [pltpu.VMEM((PB, K), jnp.int32),
           pltpu.VMEM((PB, K), jnp.int32)]
          + [pltpu.VMEM((K, cg), jnp.float32) for _ in range(NBF)]
          + [pltpu.VMEM_SHARED((acc_rows, cg), jnp.float32)]
          + ([pltpu.VMEM_SHARED((n_in, cg), jnp.float32)] if stage else [])
          + [pltpu.SemaphoreType.DMA for _ in range(NBF)]
      ),
      compiler_params=pltpu.CompilerParams(use_tc_tiling_on_sc=False),
  )
  def seg(x_hbm, src_hbm, dst_hbm, zero_hbm, out_hbm, srcb, dstb, *scr):
    rows_bufs = scr[:NBF]
    acc = scr[NBF]
    tbl = scr[NBF + 1] if stage else x_hbm
    sems = scr[(NBF + 2) if stage else (NBF + 1):]
    cid = lax.axis_index("c")
    sid = lax.axis_index("s")
    w = cid * NS + sid
    t0 = sid * rpt
    pltpu.sync_copy(zero_hbm.at[pl.ds(t0, rpt)], acc.at[pl.ds(t0, rpt)])
    if stage:
      tpt = n_in // NS
      pltpu.sync_copy(x_hbm.at[pl.ds(sid * tpt, tpt)],
                      tbl.at[pl.ds(sid * tpt, tpt)])
    plsc.subcore_barrier()

    base_row = w * n_pairs * PB

    @pl.loop(0, n_pairs)
    def _(q):
      row0 = base_row + q * PB
      pltpu.sync_copy(src_hbm.at[pl.ds(row0, PB)], srcb)
      pltpu.sync_copy(dst_hbm.at[pl.ds(row0, PB)], dstb)
      descs = {}
      for t in range(D):
        descs[t] = pltpu.async_copy(
            tbl.at[srcb.at[t]], rows_bufs[t % NBF], sems[t % NBF])
      for t in range(PB):
        descs[t].wait()
        tt = t + D
        if tt < PB:
          descs[tt] = pltpu.async_copy(
              tbl.at[srcb.at[tt]], rows_bufs[tt % NBF], sems[tt % NBF])
        pltpu.sync_copy(rows_bufs[t % NBF], acc.at[dstb.at[t]], add=True)

    plsc.subcore_barrier()
    pltpu.sync_copy(acc.at[pl.ds(t0, rpt)],
                    out_hbm.at[pl.ds(cid * acc_rows + t0, rpt)])

  return seg


def _segsum(h_groups, src, dst, n_out, acc_rows):
  cg = h_groups[0].shape[1]
  e = src.shape[0]
  _, K = _plan_cg(cg, acc_rows)
  n_raw = -(-e // (NC * NS * K))
  n_chunks, neg_ib = min((_round_up(n_raw, ib), -ib) for ib in (16, 8, 4))
  pb = -neg_ib
  total = n_chunks * NC * NS * K
  pad = total - e
  sp, dp = src, dst
  if pad:
    sp = jnp.concatenate([src, jnp.zeros((pad,), jnp.int32)])
    dp = jnp.concatenate([dst, jnp.full((pad,), n_out, jnp.int32)])
  sp = sp.reshape(n_chunks * NC * NS, K)
  dp = dp.reshape(n_chunks * NC * NS, K)
  zeros = jnp.zeros((acc_rows, cg), jnp.float32)
  n_in = h_groups[0].shape[0]
  stage = (n_in % NS == 0 and
           n_in * cg + acc_rows * cg
           + NS * (NBF * K * cg + 4 * 16 * K + 64) <= 2_040_000)
  f = _make_segsum(cg, acc_rows, K, pb, n_chunks // pb, n_in, stage)
  return [f(hg, sp, dp, zeros) for hg in h_groups]


def _mm_stats(parts, W, n_valid, acc_rows):
  cin, cout = W.shape
  G = acc_rows // BR
  ng = len(parts)

  def body(*refs):
    part_refs = refs[:2 * ng]
    w_ref, h_ref, s_ref, q_ref = refs[2 * ng:]
    i = pl.program_id(0)
    s = jnp.concatenate(
        [part_refs[2 * k][...] + part_refs[2 * k + 1][...]
         for k in range(ng)], axis=1)
    h = jnp.dot(s, w_ref[...], preferred_element_type=jnp.float32)
    h_ref[...] = h
    rows = lax.broadcasted_iota(jnp.int32, (BR, 1), 0) + i * BR
    hm = jnp.where(rows < n_valid, h, 0.0)
    ps = jnp.sum(hm, axis=0, keepdims=True)
    pq = jnp.sum(hm * hm, axis=0, keepdims=True)

    @pl.when(i == 0)
    def _():
      s_ref[...] = ps
      q_ref[...] = pq

    @pl.when(i > 0)
    def _():
      s_ref[...] += ps
      q_ref[...] += pq

  in_specs = []
  args = []
  for pk in parts:
    cg = pk.shape[1]
    in_specs.append(pl.BlockSpec((BR, cg), lambda i: (i, 0)))
    in_specs.append(
        pl.BlockSpec((BR, cg), lambda i, _G=G: (i + _G, 0)))
    args.extend([pk, pk])
  in_specs.append(pl.BlockSpec((cin, cout), lambda i: (0, 0)))
  args.append(W)

  return pl.pallas_call(
      body,
      grid=(G,),
      in_specs=in_specs,
      out_specs=[
          pl.BlockSpec((BR, cout), lambda i: (i, 0)),
          pl.BlockSpec((1, cout), lambda i: (0, 0)),
          pl.BlockSpec((1, cout), lambda i: (0, 0)),
      ],
      out_shape=[
          jax.ShapeDtypeStruct((acc_rows, cout), jnp.float32),
          jax.ShapeDtypeStruct((1, cout), jnp.float32),
          jax.ShapeDtypeStruct((1, cout), jnp.float32),
      ],
  )(*args)


def _bn_relu(H, ssum, ssq, g, b, n_valid, cg):
  acc_rows, cout = H.shape
  G = acc_rows // BR
  inv_n = 1.0 / n_valid
  ng = cout // cg

  def body(h_ref, s_ref, q_ref, g_ref, b_ref, *o_refs):
    mu = s_ref[...] * inv_n
    var = q_ref[...] * inv_n - mu * mu
    scale = g_ref[...] * lax.rsqrt(var + 1e-5)
    out = jnp.maximum((h_ref[...] - mu) * scale + b_ref[...], 0.0)
    for k in range(ng):
      o_refs[k][...] = out[:, k * cg:(k + 1) * cg]

  return pl.pallas_call(
      body,
      grid=(G,),
      in_specs=[
          pl.BlockSpec((BR, cout), lambda i: (i, 0)),
          pl.BlockSpec((1, cout), lambda i: (0, 0)),
          pl.BlockSpec((1, cout), lambda i: (0, 0)),
          pl.BlockSpec((1, cout), lambda i: (0, 0)),
          pl.BlockSpec((1, cout), lambda i: (0, 0)),
      ],
      out_specs=[pl.BlockSpec((BR, cg), lambda i: (i, 0))
                 for _ in range(ng)],
      out_shape=[jax.ShapeDtypeStruct((acc_rows, cg), jnp.float32)
                 for _ in range(ng)],
  )(H, ssum, ssq, g, b)


def _bn_gem(H, ssum, ssq, g, b, p, n_valid):
  acc_rows, cout = H.shape
  G = acc_rows // BR
  inv_n = 1.0 / n_valid

  def body(h_ref, s_ref, q_ref, g_ref, b_ref, p_ref, o_ref):
    i = pl.program_id(0)
    mu = s_ref[...] * inv_n
    var = q_ref[...] * inv_n - mu * mu
    scale = g_ref[...] * lax.rsqrt(var + 1e-5)
    h = jnp.maximum((h_ref[...] - mu) * scale + b_ref[...], 0.0)
    f = jnp.maximum(h, 1e-6)
    fp = jnp.exp(p_ref[...] * jnp.log(f))
    rows = lax.broadcasted_iota(jnp.int32, (BR, 1), 0) + i * BR
    fp = jnp.where(rows < n_valid, fp, 0.0)
    ps = jnp.sum(fp, axis=0, keepdims=True)

    @pl.when(i == 0)
    def _():
      o_ref[...] = ps

    @pl.when(i > 0)
    def _():
      o_ref[...] += ps

  return pl.pallas_call(
      body,
      grid=(G,),
      in_specs=[
          pl.BlockSpec((BR, cout), lambda i: (i, 0)),
          pl.BlockSpec((1, cout), lambda i: (0, 0)),
          pl.BlockSpec((1, cout), lambda i: (0, 0)),
          pl.BlockSpec((1, cout), lambda i: (0, 0)),
          pl.BlockSpec((1, cout), lambda i: (0, 0)),
          pl.BlockSpec((1, 1), lambda i: (0, 0)),
      ],
      out_specs=pl.BlockSpec((1, cout), lambda i: (0, 0)),
      out_shape=jax.ShapeDtypeStruct((1, cout), jnp.float32),
  )(H, ssum, ssq, g, b, p)


def _gem_final(gsum, p, n_valid):
  cout = gsum.shape[1]
  inv_n = 1.0 / n_valid

  def body(s_ref, p_ref, o_ref):
    o_ref[...] = jnp.exp(jnp.log(s_ref[...] * inv_n) / p_ref[...])

  return pl.pallas_call(
      body,
      out_shape=jax.ShapeDtypeStruct((1, cout), jnp.float32),
  )(gsum, p)


def _round_up(n, m):
  return -(-n // m) * m


def kernel(x, src0, dst0, src1a, dst1a, src1b, dst1b, src2a, dst2a, src2b,
           dst2b, src3a, dst3a, src3b, dst3b, src4a, dst4a, src4b, dst4b,
           W0, g0, b0, W1a, g1a, b1a, W1b, g1b, b1b, W2a, g2a, b2a, W2b,
           g2b, b2b, W3a, g3a, b3a, W3b, g3b, b3b, W4a, g4a, b4a, W4b,
           g4b, b4b, p):
  h_groups = [jnp.pad(x, ((0, 0), (0, 13)))]
  W0p = jnp.pad(W0, ((0, 13), (0, 0)))

  layers = [
      (src0, dst0, W0p, g0, b0, 25000),
      (src1a, dst1a, W1a, g1a, b1a, 12500),
      (src1b, dst1b, W1b, g1b, b1b, 12500),
      (src2a, dst2a, W2a, g2a, b2a, 6250),
      (src2b, dst2b, W2b, g2b, b2b, 6250),
      (src3a, dst3a, W3a, g3a, b3a, 3125),
      (src3b, dst3b, W3b, g3b, b3b, 3125),
      (src4a, dst4a, W4a, g4a, b4a, 1562),
      (src4b, dst4b, W4b, g4b, b4b, 1562),
  ]
  p2 = p.reshape(1, 1)
  gsum = None
  for li, (src, dst, W, g, b, n_out) in enumerate(layers):
    acc_rows = _round_up(n_out + 1, BR)
    parts = _segsum(h_groups, src, dst, n_out, acc_rows)
    H, ssum, ssq = _mm_stats(parts, W, n_out, acc_rows)
    g2, b2 = g.reshape(1, -1), b.reshape(1, -1)
    if li + 1 == len(layers):
      gsum = _bn_gem(H, ssum, ssq, g2, b2, p2, n_out)
    else:
      next_n_out = layers[li + 1][5]
      next_acc_rows = _round_up(next_n_out + 1, BR)
      cg_next = _plan_cg(W.shape[1], next_acc_rows)[0]
      h_groups = _bn_relu(H, ssum, ssq, g2, b2, n_out, cg_next)
  return _gem_final(gsum, p2, 1562)
